# Initial kernel scaffold; baseline (speedup 1.0000x reference)
import jax
import jax.numpy as jnp
from jax.experimental import pallas as pl


def kernel(h, edge_index, eps, W1, b1, W2, b2, gamma, beta):
    raise NotImplementedError("write your pallas kernel here")


# speedup vs baseline: 7.6230x; 7.6230x over previous
import functools

import jax
import jax.numpy as jnp
from jax import lax
from jax.experimental import pallas as pl
from jax.experimental.pallas import tpu as pltpu
from jax.experimental.pallas import tpu_sc as plsc

N, D, E = 10000, 128, 320000
NC, NS = 2, 16
NW = NC * NS
PER_W = E // NW
CHUNK = 80
NCHUNK = PER_W // CHUNK
N_PAD = 10240
ROWS_PER_SUB = N_PAD // NS

BLK = 1000


def _sc_aggregate(h, src, dst, zeros):
    mesh = plsc.VectorSubcoreMesh(core_axis_name="c", subcore_axis_name="s")

    @functools.partial(
        pl.kernel,
        out_type=jax.ShapeDtypeStruct((NC, N_PAD, D), jnp.float32),
        mesh=mesh,
        scratch_types=[
            pltpu.VMEM_SHARED((N_PAD, D), jnp.float32),
            pltpu.VMEM((NCHUNK, CHUNK), jnp.int32),
            pltpu.VMEM((NCHUNK, CHUNK), jnp.int32),
            pltpu.VMEM((CHUNK, D), jnp.float32),
        ],
    )
    def k(h_hbm, src_hbm, dst_hbm, z_hbm, out_hbm, agg_sh, src_v, dst_v, rows_v):
        c = lax.axis_index("c")
        s = lax.axis_index("s")
        row0 = s * ROWS_PER_SUB
        pltpu.sync_copy(z_hbm, agg_sh.at[pl.ds(row0, ROWS_PER_SUB)])
        pltpu.sync_copy(src_hbm.at[c, s], src_v)
        pltpu.sync_copy(dst_hbm.at[c, s], dst_v)
        plsc.subcore_barrier()

        @pl.loop(0, NCHUNK)
        def _(j):
            pltpu.sync_copy(h_hbm.at[src_v.at[j]], rows_v)
            pltpu.sync_copy(rows_v, agg_sh.at[dst_v.at[j]], add=True)

        plsc.subcore_barrier()
        pltpu.sync_copy(agg_sh.at[pl.ds(row0, ROWS_PER_SUB)],
                        out_hbm.at[c, pl.ds(row0, ROWS_PER_SUB)])

    return k(h, src, dst, zeros)


def _tc_body(eps_ref, h_ref, p_ref, w1_ref, b1_ref, w2_ref, b2_ref,
             g_ref, be_ref, o_ref):
    x = h_ref[...]
    gin = (1.0 + eps_ref[0, 0]) * x + p_ref[0] + p_ref[1]
    hid = jnp.dot(gin, w1_ref[...], preferred_element_type=jnp.float32)
    hid = jnp.maximum(hid + b1_ref[...], 0.0)
    y = jnp.dot(hid, w2_ref[...], preferred_element_type=jnp.float32)
    y = y + b2_ref[...] + x
    mu = jnp.mean(y, axis=1, keepdims=True)
    yc = y - mu
    var = jnp.mean(yc * yc, axis=1, keepdims=True)
    ynorm = yc * lax.rsqrt(var + 1e-5)
    o_ref[...] = jnp.maximum(ynorm * g_ref[...] + be_ref[...], 0.0)


def _tc_mlp(eps_arr, h, partial, W1, b1, W2, b2, gamma, beta):
    full = lambda shape: pl.BlockSpec(shape, lambda i: tuple(0 for _ in shape))
    return pl.pallas_call(
        _tc_body,
        grid=(N // BLK,),
        in_specs=[
            full((1, 1)),
            pl.BlockSpec((BLK, D), lambda i: (i, 0)),
            pl.BlockSpec((NC, BLK, D), lambda i: (0, i, 0)),
            full((D, D)),
            full((1, D)),
            full((D, D)),
            full((1, D)),
            full((1, D)),
            full((1, D)),
        ],
        out_specs=pl.BlockSpec((BLK, D), lambda i: (i, 0)),
        out_shape=jax.ShapeDtypeStruct((N, D), jnp.float32),
    )(eps_arr, h, partial, W1, b1, W2, b2, gamma, beta)


def kernel(h, edge_index, eps, W1, b1, W2, b2, gamma, beta):
    src = edge_index[0].astype(jnp.int32).reshape(NC, NS, NCHUNK, CHUNK)
    dst = edge_index[1].astype(jnp.int32).reshape(NC, NS, NCHUNK, CHUNK)
    zeros = jnp.zeros((ROWS_PER_SUB, D), jnp.float32)
    partial = _sc_aggregate(h, src, dst, zeros)
    eps_arr = eps.astype(jnp.float32).reshape(1, 1)
    return _tc_mlp(eps_arr, h, partial, W1,
                   b1.reshape(1, D), W2, b2.reshape(1, D),
                   gamma.reshape(1, D), beta.reshape(1, D))


# speedup vs baseline: 9.4498x; 1.2396x over previous
import functools

import jax
import jax.numpy as jnp
from jax import lax
from jax.experimental import pallas as pl
from jax.experimental.pallas import tpu as pltpu
from jax.experimental.pallas import tpu_sc as plsc

N, D, E = 10000, 128, 320000
NC, NS = 2, 16
NW = NC * NS
PER_W = E // NW
CHUNK = 40
NCHUNK = PER_W // CHUNK
SUP = 25
NSUP = NCHUNK // SUP
N_PAD = 10112
ROWS_PER_SUB = N_PAD // NS

BLK = 1000


def _sc_aggregate(h, edges, zeros):
    mesh = plsc.VectorSubcoreMesh(core_axis_name="c", subcore_axis_name="s")

    @functools.partial(
        pl.kernel,
        out_type=jax.ShapeDtypeStruct((NC, N_PAD, D), jnp.float32),
        mesh=mesh,
        scratch_types=[
            pltpu.VMEM_SHARED((N_PAD, D), jnp.float32),
            pltpu.VMEM((2, SUP, CHUNK), jnp.int32),
            pltpu.VMEM((2, SUP, CHUNK), jnp.int32),
            pltpu.VMEM((2, CHUNK, D), jnp.float32),
            pltpu.SemaphoreType.DMA((2,)),
            pltpu.SemaphoreType.DMA((2,)),
        ],
    )
    def k(h_hbm, e_hbm, z_hbm, out_hbm, agg_sh, src_v, dst_v, rows, gsem, isem):
        c = lax.axis_index("c")
        s = lax.axis_index("s")
        row0 = s * ROWS_PER_SUB
        pltpu.sync_copy(z_hbm, agg_sh.at[pl.ds(row0, ROWS_PER_SUB)])

        def start_idx(u, up):
            pltpu.async_copy(e_hbm.at[0, c, s, u], src_v.at[up], isem.at[up])
            pltpu.async_copy(e_hbm.at[1, c, s, u], dst_v.at[up], isem.at[up])

        def wait_idx(u, up):
            pltpu.make_async_copy(e_hbm.at[0, c, s, u], src_v.at[up],
                                  isem.at[up]).wait()
            pltpu.make_async_copy(e_hbm.at[1, c, s, u], dst_v.at[up],
                                  isem.at[up]).wait()

        @pl.loop(0, 2)
        def _(u):
            start_idx(u, u)

        plsc.subcore_barrier()

        @pl.loop(0, NSUP)
        def _(u):
            up = u & 1
            wait_idx(u, up)

            @pl.loop(0, SUP + 2)
            def _(jj):
                @pl.when(jj >= 2)
                def _():
                    kk = jj - 2
                    p = kk & 1
                    pltpu.make_async_copy(h_hbm.at[src_v.at[up, kk]],
                                          rows.at[p], gsem.at[p]).wait()
                    pltpu.sync_copy(rows.at[p], agg_sh.at[dst_v.at[up, kk]],
                                    add=True)

                @pl.when(jj < SUP)
                def _():
                    pltpu.async_copy(h_hbm.at[src_v.at[up, jj]],
                                     rows.at[jj & 1], gsem.at[jj & 1])

            @pl.when(u + 2 < NSUP)
            def _():
                start_idx(u + 2, up)

        plsc.subcore_barrier()
        pltpu.sync_copy(agg_sh.at[pl.ds(row0, ROWS_PER_SUB)],
                        out_hbm.at[c, pl.ds(row0, ROWS_PER_SUB)])

    return k(h, edges, zeros)


def _tc_body(eps_ref, h_ref, p_ref, w1_ref, b1_ref, w2_ref, b2_ref,
             g_ref, be_ref, o_ref):
    x = h_ref[...]
    gin = (1.0 + eps_ref[0, 0]) * x + p_ref[0] + p_ref[1]
    hid = jnp.dot(gin, w1_ref[...], preferred_element_type=jnp.float32)
    hid = jnp.maximum(hid + b1_ref[...], 0.0)
    y = jnp.dot(hid, w2_ref[...], preferred_element_type=jnp.float32)
    y = y + b2_ref[...] + x
    mu = jnp.mean(y, axis=1, keepdims=True)
    yc = y - mu
    var = jnp.mean(yc * yc, axis=1, keepdims=True)
    ynorm = yc * lax.rsqrt(var + 1e-5)
    o_ref[...] = jnp.maximum(ynorm * g_ref[...] + be_ref[...], 0.0)


def _tc_mlp(eps_arr, h, partial, W1, b1, W2, b2, gamma, beta):
    full = lambda shape: pl.BlockSpec(shape, lambda i: tuple(0 for _ in shape))
    return pl.pallas_call(
        _tc_body,
        grid=(N // BLK,),
        in_specs=[
            full((1, 1)),
            pl.BlockSpec((BLK, D), lambda i: (i, 0)),
            pl.BlockSpec((NC, BLK, D), lambda i: (0, i, 0)),
            full((D, D)),
            full((1, D)),
            full((D, D)),
            full((1, D)),
            full((1, D)),
            full((1, D)),
        ],
        out_specs=pl.BlockSpec((BLK, D), lambda i: (i, 0)),
        out_shape=jax.ShapeDtypeStruct((N, D), jnp.float32),
    )(eps_arr, h, partial, W1, b1, W2, b2, gamma, beta)


def kernel(h, edge_index, eps, W1, b1, W2, b2, gamma, beta):
    edges = edge_index.astype(jnp.int32).reshape(2, NC, NS, NSUP, SUP, CHUNK)
    zeros = jnp.zeros((ROWS_PER_SUB, D), jnp.float32)
    partial = _sc_aggregate(h, edges, zeros)
    eps_arr = eps.astype(jnp.float32).reshape(1, 1)
    return _tc_mlp(eps_arr, h, partial, W1,
                   b1.reshape(1, D), W2, b2.reshape(1, D),
                   gamma.reshape(1, D), beta.reshape(1, D))


# speedup vs baseline: 9.9141x; 1.0491x over previous
import functools

import jax
import jax.numpy as jnp
from jax import lax
from jax.experimental import pallas as pl
from jax.experimental.pallas import tpu as pltpu
from jax.experimental.pallas import tpu_sc as plsc

N, D, E = 10000, 128, 320000
NC, NS = 2, 16
NW = NC * NS
CHUNK = 32
PER_W = 10240
NCHUNK = PER_W // CHUNK
SUP = 16
NSUP = NCHUNK // SUP
NBUF = 3
N_PAD = 10112
ROWS_PER_SUB = N_PAD // NS
E_PAD = NW * PER_W

BLK = 1000


def _sc_aggregate(h, edges, zeros):
    mesh = plsc.VectorSubcoreMesh(core_axis_name="c", subcore_axis_name="s")

    @functools.partial(
        pl.kernel,
        out_type=jax.ShapeDtypeStruct((NC, N_PAD, D), jnp.float32),
        mesh=mesh,
        scratch_types=[
            pltpu.VMEM_SHARED((N_PAD, D), jnp.float32),
            pltpu.VMEM((NBUF, SUP, CHUNK), jnp.int32),
            pltpu.VMEM((NBUF, SUP, CHUNK), jnp.int32),
            pltpu.VMEM((NBUF, CHUNK, D), jnp.float32),
            pltpu.SemaphoreType.DMA((NBUF,)),
            pltpu.SemaphoreType.DMA((NBUF,)),
        ],
    )
    def k(h_hbm, e_hbm, z_hbm, out_hbm, agg_sh, src_v, dst_v, rows, gsem, isem):
        c = lax.axis_index("c")
        s = lax.axis_index("s")
        row0 = s * ROWS_PER_SUB
        pltpu.sync_copy(z_hbm, agg_sh.at[pl.ds(row0, ROWS_PER_SUB)])

        def start_idx(u, up):
            pltpu.async_copy(e_hbm.at[0, c, s, u], src_v.at[up], isem.at[up])
            pltpu.async_copy(e_hbm.at[1, c, s, u], dst_v.at[up], isem.at[up])

        def wait_idx(u, up):
            pltpu.make_async_copy(e_hbm.at[0, c, s, u], src_v.at[up],
                                  isem.at[up]).wait()
            pltpu.make_async_copy(e_hbm.at[1, c, s, u], dst_v.at[up],
                                  isem.at[up]).wait()

        @pl.loop(0, 2)
        def _(u):
            start_idx(u, u % NBUF)

        plsc.subcore_barrier()

        @pl.loop(0, NCHUNK + 2)
        def _(j):
            @pl.when(j < NCHUNK)
            def _():
                u = j >> 4
                jj = j & (SUP - 1)
                up = u % NBUF

                @pl.when(jj == 0)
                def _():
                    wait_idx(u, up)

                pltpu.async_copy(h_hbm.at[src_v.at[up, jj]],
                                 rows.at[j % NBUF], gsem.at[j % NBUF])

                @pl.when(jnp.logical_and(jj == 2, u + 2 < NSUP))
                def _():
                    start_idx(u + 2, (u + 2) % NBUF)

            @pl.when(j >= 2)
            def _():
                k2 = j - 2
                uk = k2 >> 4
                kk = k2 & (SUP - 1)
                ukp = uk % NBUF
                p = k2 % NBUF
                pltpu.make_async_copy(h_hbm.at[src_v.at[ukp, kk]],
                                      rows.at[p], gsem.at[p]).wait()
                pltpu.sync_copy(rows.at[p], agg_sh.at[dst_v.at[ukp, kk]],
                                add=True)

        plsc.subcore_barrier()
        pltpu.sync_copy(agg_sh.at[pl.ds(row0, ROWS_PER_SUB)],
                        out_hbm.at[c, pl.ds(row0, ROWS_PER_SUB)])

    return k(h, edges, zeros)


def _tc_body(eps_ref, h_ref, p_ref, w1_ref, b1_ref, w2_ref, b2_ref,
             g_ref, be_ref, o_ref):
    x = h_ref[...]
    gin = (1.0 + eps_ref[0, 0]) * x + p_ref[0] + p_ref[1]
    hid = jnp.dot(gin, w1_ref[...], preferred_element_type=jnp.float32)
    hid = jnp.maximum(hid + b1_ref[...], 0.0)
    y = jnp.dot(hid, w2_ref[...], preferred_element_type=jnp.float32)
    y = y + b2_ref[...] + x
    mu = jnp.mean(y, axis=1, keepdims=True)
    yc = y - mu
    var = jnp.mean(yc * yc, axis=1, keepdims=True)
    ynorm = yc * lax.rsqrt(var + 1e-5)
    o_ref[...] = jnp.maximum(ynorm * g_ref[...] + be_ref[...], 0.0)


def _tc_mlp(eps_arr, h, partial, W1, b1, W2, b2, gamma, beta):
    full = lambda shape: pl.BlockSpec(shape, lambda i: tuple(0 for _ in shape))
    return pl.pallas_call(
        _tc_body,
        grid=(N // BLK,),
        in_specs=[
            full((1, 1)),
            pl.BlockSpec((BLK, D), lambda i: (i, 0)),
            pl.BlockSpec((NC, BLK, D), lambda i: (0, i, 0)),
            full((D, D)),
            full((1, D)),
            full((D, D)),
            full((1, D)),
            full((1, D)),
            full((1, D)),
        ],
        out_specs=pl.BlockSpec((BLK, D), lambda i: (i, 0)),
        out_shape=jax.ShapeDtypeStruct((N, D), jnp.float32),
    )(eps_arr, h, partial, W1, b1, W2, b2, gamma, beta)


def kernel(h, edge_index, eps, W1, b1, W2, b2, gamma, beta):
    npad = E_PAD - E
    pad_iota = jnp.arange(npad, dtype=jnp.int32)
    ei = edge_index.astype(jnp.int32)
    src_full = jnp.concatenate([ei[0], pad_iota % N])
    dst_full = jnp.concatenate([ei[1], N + pad_iota % (N_PAD - N)])
    edges = jnp.stack([src_full, dst_full]).reshape(2, NC, NS, NSUP, SUP, CHUNK)
    zeros = jnp.zeros((ROWS_PER_SUB, D), jnp.float32)
    partial = _sc_aggregate(h, edges, zeros)
    eps_arr = eps.astype(jnp.float32).reshape(1, 1)
    return _tc_mlp(eps_arr, h, partial, W1,
                   b1.reshape(1, D), W2, b2.reshape(1, D),
                   gamma.reshape(1, D), beta.reshape(1, D))


# speedup vs baseline: 12.1840x; 1.2289x over previous
import functools

import jax
import jax.numpy as jnp
from jax import lax
from jax.experimental import pallas as pl
from jax.experimental.pallas import tpu as pltpu
from jax.experimental.pallas import tpu_sc as plsc

N, D, E = 10000, 128, 320000
NC, NS = 2, 16
NW = NC * NS
CHUNK = 40
PER_W = E // NW
NCHUNK = PER_W // CHUNK
SUP = 2
SUP_SH = 1
NSUP = NCHUNK // SUP
NBUF = 3
N_PAD = 10112
ROWS_PER_SUB = N_PAD // NS

BLK = 2000


def _sc_aggregate(h, edges):
    mesh = plsc.VectorSubcoreMesh(core_axis_name="c", subcore_axis_name="s")

    @functools.partial(
        pl.kernel,
        out_type=jax.ShapeDtypeStruct((NC, N_PAD, D), jnp.float32),
        mesh=mesh,
        scratch_types=[
            pltpu.VMEM_SHARED((N_PAD, D), jnp.float32),
            pltpu.VMEM((NBUF, SUP, CHUNK), jnp.int32),
            pltpu.VMEM((NBUF, SUP, CHUNK), jnp.int32),
            pltpu.VMEM((NBUF, CHUNK, D), jnp.float32),
            pltpu.SemaphoreType.DMA((NBUF,)),
            pltpu.SemaphoreType.DMA((NBUF,)),
        ],
    )
    def k(h_hbm, e_hbm, out_hbm, agg_sh, src_v, dst_v, rows, gsem, isem):
        c = lax.axis_index("c")
        s = lax.axis_index("s")
        row0 = s * ROWS_PER_SUB

        def start_idx(u, up):
            pltpu.async_copy(e_hbm.at[0, c, s, u], src_v.at[up], isem.at[up])
            pltpu.async_copy(e_hbm.at[1, c, s, u], dst_v.at[up], isem.at[up])

        def wait_idx(u, up):
            pltpu.make_async_copy(e_hbm.at[0, c, s, u], src_v.at[up],
                                  isem.at[up]).wait()
            pltpu.make_async_copy(e_hbm.at[1, c, s, u], dst_v.at[up],
                                  isem.at[up]).wait()

        @pl.loop(0, 2)
        def _(u):
            start_idx(u, u % NBUF)

        @pl.loop(0, NBUF * CHUNK)
        def _(r):
            @pl.loop(0, D // 16)
            def _(c16):
                rows[r // CHUNK, r % CHUNK, pl.ds(c16 * 16, 16)] = (
                    jnp.zeros((16,), jnp.float32))

        @pl.loop(0, ROWS_PER_SUB // CHUNK)
        def _(i):
            pltpu.sync_copy(rows.at[0],
                            agg_sh.at[pl.ds(row0 + i * CHUNK, CHUNK)])

        @pl.when((ROWS_PER_SUB % CHUNK) > 0)
        def _():
            rem = ROWS_PER_SUB % CHUNK
            pltpu.sync_copy(
                rows.at[0, pl.ds(0, rem)],
                agg_sh.at[pl.ds(row0 + (ROWS_PER_SUB // CHUNK) * CHUNK, rem)])

        plsc.subcore_barrier()

        @pl.loop(0, NCHUNK + 2)
        def _(j):
            u = j >> SUP_SH
            jj = j & (SUP - 1)

            @pl.when(j < NCHUNK)
            def _():
                up = u % NBUF

                @pl.when(jj == 0)
                def _():
                    wait_idx(u, up)

                pltpu.async_copy(h_hbm.at[src_v.at[up, jj]],
                                 rows.at[j % NBUF], gsem.at[j % NBUF])

            @pl.when(j >= 2)
            def _():
                k2 = j - 2
                uk = k2 >> SUP_SH
                kk = k2 & (SUP - 1)
                ukp = uk % NBUF
                p = k2 % NBUF
                pltpu.make_async_copy(h_hbm.at[src_v.at[ukp, kk]],
                                      rows.at[p], gsem.at[p]).wait()
                pltpu.sync_copy(rows.at[p], agg_sh.at[dst_v.at[ukp, kk]],
                                add=True)

            @pl.when(jnp.logical_and(j < NCHUNK,
                                     jnp.logical_and(jj == SUP - 1,
                                                     u + 2 < NSUP)))
            def _():
                start_idx(u + 2, (u + 2) % NBUF)

        plsc.subcore_barrier()
        pltpu.sync_copy(agg_sh.at[pl.ds(row0, ROWS_PER_SUB)],
                        out_hbm.at[c, pl.ds(row0, ROWS_PER_SUB)])

    return k(h, edges)


def _tc_body(eps_ref, h_ref, p_ref, w1_ref, b1_ref, w2_ref, b2_ref,
             g_ref, be_ref, o_ref):
    x = h_ref[...]
    gin = (1.0 + eps_ref[0, 0]) * x + p_ref[0] + p_ref[1]
    hid = jnp.dot(gin, w1_ref[...], preferred_element_type=jnp.float32)
    hid = jnp.maximum(hid + b1_ref[...], 0.0)
    y = jnp.dot(hid, w2_ref[...], preferred_element_type=jnp.float32)
    y = y + b2_ref[...] + x
    mu = jnp.mean(y, axis=1, keepdims=True)
    yc = y - mu
    var = jnp.mean(yc * yc, axis=1, keepdims=True)
    ynorm = yc * lax.rsqrt(var + 1e-5)
    o_ref[...] = jnp.maximum(ynorm * g_ref[...] + be_ref[...], 0.0)


def _tc_mlp(eps_arr, h, partial, W1, b1, W2, b2, gamma, beta):
    full = lambda shape: pl.BlockSpec(shape, lambda i: tuple(0 for _ in shape))
    return pl.pallas_call(
        _tc_body,
        grid=(N // BLK,),
        in_specs=[
            full((1, 1)),
            pl.BlockSpec((BLK, D), lambda i: (i, 0)),
            pl.BlockSpec((NC, BLK, D), lambda i: (0, i, 0)),
            full((D, D)),
            full((1, D)),
            full((D, D)),
            full((1, D)),
            full((1, D)),
            full((1, D)),
        ],
        out_specs=pl.BlockSpec((BLK, D), lambda i: (i, 0)),
        out_shape=jax.ShapeDtypeStruct((N, D), jnp.float32),
    )(eps_arr, h, partial, W1, b1, W2, b2, gamma, beta)


def kernel(h, edge_index, eps, W1, b1, W2, b2, gamma, beta):
    edges = edge_index.astype(jnp.int32).reshape(2, NC, NS, NSUP, SUP, CHUNK)
    partial = _sc_aggregate(h, edges)
    eps_arr = eps.astype(jnp.float32).reshape(1, 1)
    return _tc_mlp(eps_arr, h, partial, W1,
                   b1.reshape(1, D), W2, b2.reshape(1, D),
                   gamma.reshape(1, D), beta.reshape(1, D))
